# Initial kernel scaffold; baseline (speedup 1.0000x reference)
import jax
import jax.numpy as jnp
from jax.experimental import pallas as pl


def kernel(x, W_enc_lips, W_enc_exp, W_enc_rest, W_enc_rot, W_dec_lips, W_dec_exp, W_dec_rest, W_dec_rot):
    raise NotImplementedError("write your pallas kernel here")


# speedup vs baseline: 1.2071x; 1.2071x over previous
import jax
import jax.numpy as jnp
from jax.experimental import pallas as pl
from jax.experimental.pallas import tpu as pltpu

_L_LIPS, _D_LIPS = 8, 5
_L_EXP, _D_EXP = 8, 5
_L_REST, _D_REST = 5, 6
_L_ROT, _D_ROT = 7, 4
_OFF_LIPS = 0
_OFF_EXP = _OFF_LIPS + _L_LIPS ** _D_LIPS
_OFF_REST = _OFF_EXP + _L_EXP ** _D_EXP
_OFF_ROT = _OFF_REST + _L_REST ** _D_REST

_F = 205
_DQ = 20
_BT = 1024


def _body(x_ref, wall_ref, wout_ref, half_ref, pow_ref, out_ref, codes_ref):
    xb = x_ref[...]
    half = half_ref[...]
    z = jnp.dot(xb, wall_ref[...], preferred_element_type=jnp.float32)
    q = jnp.round(half * jnp.tanh(z))
    out_ref[...] = jnp.dot(q, wout_ref[...], preferred_element_type=jnp.float32)
    t = (q + jnp.floor(half)) * pow_ref[...]
    c_lips = jnp.sum(t[:, 0:5], axis=-1, keepdims=True)
    c_exp = jnp.sum(t[:, 5:10], axis=-1, keepdims=True)
    c_rest = jnp.sum(t[:, 10:16], axis=-1, keepdims=True)
    c_rot = jnp.sum(t[:, 16:20], axis=-1, keepdims=True)
    codes_f = jnp.concatenate([
        c_lips + float(_OFF_LIPS),
        c_exp + float(_OFF_EXP),
        c_rest + float(_OFF_REST),
        c_rot + float(_OFF_ROT),
    ], axis=-1)
    codes_ref[...] = codes_f.astype(jnp.int32)


def kernel(x, W_enc_lips, W_enc_exp, W_enc_rest, W_enc_rot,
           W_dec_lips, W_dec_exp, W_dec_rest, W_dec_rot):
    B, T, F = x.shape
    N = B * T
    x2 = x.reshape(N, F)

    f32 = jnp.float32
    W_all = jnp.zeros((F, _DQ), f32)
    W_all = W_all.at[12:75, 0:5].set(W_enc_lips)
    W_all = W_all.at[12:75, 5:10].set(W_enc_exp)
    W_all = W_all.at[75:205, 10:16].set(W_enc_rest)
    W_all = W_all.at[0:12, 16:20].set(W_enc_rot)

    W_out = jnp.zeros((_DQ, F), f32)
    W_out = W_out.at[0:5, 60:75].set(W_dec_lips)
    W_out = W_out.at[5:10, 12:60].set(W_dec_exp)
    W_out = W_out.at[10:16, 9:12].set(W_dec_rest[:, 0:3])
    W_out = W_out.at[10:16, 75:138].set(W_dec_rest[:, 3:66])
    W_out = W_out.at[10:16, 139:142].set(W_dec_rest[:, 66:69])
    W_out = W_out.at[10:16, 142:205].set(W_dec_rest[:, 69:132])
    W_out = W_out.at[16:20, 0:9].set(W_dec_rot[:, 0:9])
    W_out = W_out.at[16:20, 138:139].set(W_dec_rot[:, 9:10])

    halves = jnp.array([[ (_L_LIPS - 1) / 2.0 ] * 5
                        + [ (_L_EXP - 1) / 2.0 ] * 5
                        + [ (_L_REST - 1) / 2.0 ] * 6
                        + [ (_L_ROT - 1) / 2.0 ] * 4], f32)
    powers = jnp.array([[float(_L_LIPS ** i) for i in range(5)]
                        + [float(_L_EXP ** i) for i in range(5)]
                        + [float(_L_REST ** i) for i in range(6)]
                        + [float(_L_ROT ** i) for i in range(4)]], f32)

    grid = (N // _BT,)
    out2, codes2 = pl.pallas_call(
        _body,
        grid=grid,
        in_specs=[
            pl.BlockSpec((_BT, F), lambda i: (i, 0)),
            pl.BlockSpec((F, _DQ), lambda i: (0, 0)),
            pl.BlockSpec((_DQ, F), lambda i: (0, 0)),
            pl.BlockSpec((1, _DQ), lambda i: (0, 0)),
            pl.BlockSpec((1, _DQ), lambda i: (0, 0)),
        ],
        out_specs=[
            pl.BlockSpec((_BT, F), lambda i: (i, 0)),
            pl.BlockSpec((_BT, 4), lambda i: (i, 0)),
        ],
        out_shape=[
            jax.ShapeDtypeStruct((N, F), f32),
            jax.ShapeDtypeStruct((N, 4), jnp.int32),
        ],
        compiler_params=pltpu.CompilerParams(
            dimension_semantics=("arbitrary",),
        ),
    )(x2, W_all, W_out, halves, powers)

    out = out2.reshape(B, T, F)
    codes = codes2.T.reshape(4, B, T)
    return out, codes


# speedup vs baseline: 1.3413x; 1.1112x over previous
import jax
import jax.numpy as jnp
from jax.experimental import pallas as pl
from jax.experimental.pallas import tpu as pltpu

_L_LIPS = 8
_L_EXP = 8
_L_REST = 5
_L_ROT = 7
_OFF_LIPS = 0
_OFF_EXP = _OFF_LIPS + _L_LIPS ** 5
_OFF_REST = _OFF_EXP + _L_EXP ** 5
_OFF_ROT = _OFF_REST + _L_REST ** 6

_F = 205
_DQ = 20
_FC = _F + 4
_BT = 1024


def _body(x_ref, wall_ref, wcat_ref, half_ref, c_ref, out_ref, codes_ref):
    xb = x_ref[0]
    z = jnp.dot(xb, wall_ref[...], preferred_element_type=jnp.float32)
    q = jnp.round(half_ref[...] * jnp.tanh(z))
    y = jnp.dot(q, wcat_ref[...], preferred_element_type=jnp.float32)
    out_ref[0] = y[:, :_F]
    codes_ref[0] = (y[:, _F:_FC] + c_ref[...]).astype(jnp.int32)


def kernel(x, W_enc_lips, W_enc_exp, W_enc_rest, W_enc_rot,
           W_dec_lips, W_dec_exp, W_dec_rest, W_dec_rot):
    B, T, F = x.shape

    f32 = jnp.float32
    W_all = jnp.zeros((F, _DQ), f32)
    W_all = W_all.at[12:75, 0:5].set(W_enc_lips)
    W_all = W_all.at[12:75, 5:10].set(W_enc_exp)
    W_all = W_all.at[75:205, 10:16].set(W_enc_rest)
    W_all = W_all.at[0:12, 16:20].set(W_enc_rot)

    W_cat = jnp.zeros((_DQ, _FC), f32)
    W_cat = W_cat.at[0:5, 60:75].set(W_dec_lips)
    W_cat = W_cat.at[5:10, 12:60].set(W_dec_exp)
    W_cat = W_cat.at[10:16, 9:12].set(W_dec_rest[:, 0:3])
    W_cat = W_cat.at[10:16, 75:138].set(W_dec_rest[:, 3:66])
    W_cat = W_cat.at[10:16, 139:142].set(W_dec_rest[:, 66:69])
    W_cat = W_cat.at[10:16, 142:205].set(W_dec_rest[:, 69:132])
    W_cat = W_cat.at[16:20, 0:9].set(W_dec_rot[:, 0:9])
    W_cat = W_cat.at[16:20, 138:139].set(W_dec_rot[:, 9:10])
    W_cat = W_cat.at[0:5, 205].set(jnp.array([float(_L_LIPS ** i) for i in range(5)], f32))
    W_cat = W_cat.at[5:10, 206].set(jnp.array([float(_L_EXP ** i) for i in range(5)], f32))
    W_cat = W_cat.at[10:16, 207].set(jnp.array([float(_L_REST ** i) for i in range(6)], f32))
    W_cat = W_cat.at[16:20, 208].set(jnp.array([float(_L_ROT ** i) for i in range(4)], f32))

    halves = jnp.array([[3.5] * 10 + [2.0] * 6 + [3.0] * 4], f32)
    c_vec = jnp.array([[
        _OFF_LIPS + 3.0 * sum(_L_LIPS ** i for i in range(5)),
        _OFF_EXP + 3.0 * sum(_L_EXP ** i for i in range(5)),
        _OFF_REST + 2.0 * sum(_L_REST ** i for i in range(6)),
        _OFF_ROT + 3.0 * sum(_L_ROT ** i for i in range(4)),
    ]], f32)

    grid = (B, T // _BT)
    out, codes3 = pl.pallas_call(
        _body,
        grid=grid,
        in_specs=[
            pl.BlockSpec((1, _BT, F), lambda b, t: (b, t, 0)),
            pl.BlockSpec((F, _DQ), lambda b, t: (0, 0)),
            pl.BlockSpec((_DQ, _FC), lambda b, t: (0, 0)),
            pl.BlockSpec((1, _DQ), lambda b, t: (0, 0)),
            pl.BlockSpec((1, 4), lambda b, t: (0, 0)),
        ],
        out_specs=[
            pl.BlockSpec((1, _BT, F), lambda b, t: (b, t, 0)),
            pl.BlockSpec((1, _BT, 4), lambda b, t: (b, t, 0)),
        ],
        out_shape=[
            jax.ShapeDtypeStruct((B, T, F), f32),
            jax.ShapeDtypeStruct((B, T, 4), jnp.int32),
        ],
        compiler_params=pltpu.CompilerParams(
            dimension_semantics=("arbitrary", "arbitrary"),
        ),
    )(x, W_all, W_cat, halves, c_vec)

    codes = codes3.transpose(2, 0, 1)
    return out, codes


# speedup vs baseline: 1.8246x; 1.3603x over previous
import jax
import jax.numpy as jnp
from jax import lax
from jax.experimental import pallas as pl
from jax.experimental.pallas import tpu as pltpu

_L_LIPS = 8
_L_EXP = 8
_L_REST = 5
_L_ROT = 7
_OFF_LIPS = 0
_OFF_EXP = _OFF_LIPS + _L_LIPS ** 5
_OFF_REST = _OFF_EXP + _L_EXP ** 5
_OFF_ROT = _OFF_REST + _L_REST ** 6

_F = 205
_DQ = 20
_BT = 1024


def _body(x_ref, wl_ref, we_ref, wr_ref, wo_ref, dl_ref, de_ref, dr_ref,
          do_ref, half_ref, pt_ref, c_ref, out_ref, codes_ref, wall_s, wout_s):
    f32 = jnp.float32
    step = pl.program_id(0) * pl.num_programs(1) + pl.program_id(1)

    @pl.when(step == 0)
    def _assemble():
        z = lambda r, c: jnp.zeros((r, c), f32)
        col_le = jnp.concatenate(
            [z(12, 10),
             jnp.concatenate([wl_ref[...], we_ref[...]], axis=1),
             z(130, 10)], axis=0)
        col_rest = jnp.concatenate([z(75, 6), wr_ref[...]], axis=0)
        col_rot = jnp.concatenate([wo_ref[...], z(193, 4)], axis=0)
        wall_s[...] = jnp.concatenate([col_le, col_rest, col_rot], axis=1)
        dl, de, dr, do = dl_ref[...], de_ref[...], dr_ref[...], do_ref[...]
        r_lips = jnp.concatenate([z(5, 60), dl, z(5, 130)], axis=1)
        r_exp = jnp.concatenate([z(5, 12), de, z(5, 145)], axis=1)
        r_rest = jnp.concatenate(
            [z(6, 9), dr[:, 0:3], z(6, 63), dr[:, 3:66], z(6, 1),
             dr[:, 66:69], dr[:, 69:132]], axis=1)
        r_rot = jnp.concatenate(
            [do[:, 0:9], z(4, 129), do[:, 9:10], z(4, 66)], axis=1)
        wout_s[...] = jnp.concatenate([r_lips, r_exp, r_rest, r_rot], axis=0)

    half = half_ref[...]
    xb = x_ref[0]
    zq = jnp.dot(xb, wall_s[...], preferred_element_type=f32)
    q = jnp.round(half * jnp.tanh(zq))
    out_ref[0] = jnp.dot(q, wout_s[...], preferred_element_type=f32)

    codes_t = lax.dot_general(pt_ref[...], q, (((1,), (1,)), ((), ())),
                              preferred_element_type=f32)
    codes_ref[...] = (codes_t + c_ref[...]).astype(jnp.int32)


def kernel(x, W_enc_lips, W_enc_exp, W_enc_rest, W_enc_rot,
           W_dec_lips, W_dec_exp, W_dec_rest, W_dec_rot):
    B, T, F = x.shape
    TB = T // _BT
    f32 = jnp.float32

    half = jnp.array([[3.5] * 10 + [2.0] * 6 + [3.0] * 4], f32)
    pt = jnp.array(
        [[float(_L_LIPS ** i) for i in range(5)] + [0.0] * 15,
         [0.0] * 5 + [float(_L_EXP ** i) for i in range(5)] + [0.0] * 10,
         [0.0] * 10 + [float(_L_REST ** i) for i in range(6)] + [0.0] * 4,
         [0.0] * 16 + [float(_L_ROT ** i) for i in range(4)]], f32)
    c_vec = jnp.array(
        [[_OFF_LIPS + 3.0 * sum(_L_LIPS ** i for i in range(5))],
         [_OFF_EXP + 3.0 * sum(_L_EXP ** i for i in range(5))],
         [_OFF_REST + 2.0 * sum(_L_REST ** i for i in range(6))],
         [_OFF_ROT + 3.0 * sum(_L_ROT ** i for i in range(4))]], f32)

    grid = (B, TB)
    full = lambda shape: pl.BlockSpec(shape, lambda b, t: tuple(0 for _ in shape))
    out, codes2 = pl.pallas_call(
        _body,
        grid=grid,
        in_specs=[
            pl.BlockSpec((1, _BT, F), lambda b, t: (b, t, 0)),
            full((63, 5)), full((63, 5)), full((130, 6)), full((12, 4)),
            full((5, 15)), full((5, 48)), full((6, 132)), full((4, 10)),
            full((1, _DQ)), full((4, _DQ)), full((4, 1)),
        ],
        out_specs=[
            pl.BlockSpec((1, _BT, F), lambda b, t: (b, t, 0)),
            pl.BlockSpec((4, _BT), lambda b, t: (0, b * TB + t)),
        ],
        out_shape=[
            jax.ShapeDtypeStruct((B, T, F), f32),
            jax.ShapeDtypeStruct((4, B * T), jnp.int32),
        ],
        scratch_shapes=[
            pltpu.VMEM((F, _DQ), f32),
            pltpu.VMEM((_DQ, F), f32),
        ],
        compiler_params=pltpu.CompilerParams(
            dimension_semantics=("arbitrary", "arbitrary"),
        ),
    )(x, W_enc_lips, W_enc_exp, W_enc_rest, W_enc_rot,
      W_dec_lips, W_dec_exp, W_dec_rest, W_dec_rot, half, pt, c_vec)

    return out, codes2.reshape(4, B, T)
